# Initial kernel scaffold; baseline (speedup 1.0000x reference)
import jax
import jax.numpy as jnp
from jax.experimental import pallas as pl


def kernel(x, edge_index, W1, b1, W2, b2, W3, b3, W4, b4, W5, b5):
    raise NotImplementedError("write your pallas kernel here")


# speedup vs baseline: 8.2811x; 8.2811x over previous
import functools

import jax
import jax.numpy as jnp
from jax import lax
from jax.experimental import pallas as pl
from jax.experimental.pallas import tpu as pltpu
from jax.experimental.pallas import tpu_sc as plsc

N_NODES = 10000
N_EDGES = 320000
IN_DIM = 128
HID = 64

NC, NS, LANES = 2, 16, 16
NW = NC * NS
EPB = 128
BPT = 80
EPT = EPB * BPT
ETOT = EPT * NW
NPAD = 10240
RPT = NPAD // NS

_MESH = plsc.VectorSubcoreMesh(core_axis_name="c", subcore_axis_name="s")


def _sc_degree_body(dst_hbm, out_hbm, idx_v, ones_v, zrow_v, acc_sh):
    cid = lax.axis_index("c")
    sid = lax.axis_index("s")
    wid = cid * NS + sid
    base = sid * RPT

    def fill(i, _):
        zrow_v[i] = jnp.zeros((LANES,), jnp.float32)
        ones_v[i] = jnp.ones((LANES,), jnp.float32)
        return 0

    lax.fori_loop(0, EPB, fill, 0)
    for k in range(RPT // EPB):
        pltpu.sync_copy(zrow_v, acc_sh.at[pl.ds(base + k * EPB, EPB)])
    plsc.subcore_barrier()

    pltpu.sync_copy(dst_hbm.at[pl.ds(wid * BPT, BPT)], idx_v)

    def body(j, _):
        pltpu.sync_copy(ones_v, acc_sh.at[idx_v.at[j]], add=True)
        return 0

    lax.fori_loop(0, BPT, body, 0)
    plsc.subcore_barrier()
    pltpu.sync_copy(acc_sh.at[pl.ds(base, RPT)],
                    out_hbm.at[cid, pl.ds(base, RPT)])


def _sc_spmm_body(tbl_hbm, src_hbm, dst_hbm, out_hbm,
                  sidx_v, didx_v, rows_v, zrow_v, acc_sh):
    cid = lax.axis_index("c")
    sid = lax.axis_index("s")
    wid = cid * NS + sid
    base = sid * RPT

    def fill(i, _):
        for k in range(HID // LANES):
            zrow_v[i, pl.ds(k * LANES, LANES)] = jnp.zeros((LANES,), jnp.float32)
        return 0

    lax.fori_loop(0, EPB, fill, 0)
    for k in range(RPT // EPB):
        pltpu.sync_copy(zrow_v, acc_sh.at[pl.ds(base + k * EPB, EPB)])
    plsc.subcore_barrier()

    pltpu.sync_copy(src_hbm.at[pl.ds(wid * BPT, BPT)], sidx_v)
    pltpu.sync_copy(dst_hbm.at[pl.ds(wid * BPT, BPT)], didx_v)

    def body(j, _):
        pltpu.sync_copy(tbl_hbm.at[sidx_v.at[j]], rows_v)
        pltpu.sync_copy(rows_v, acc_sh.at[didx_v.at[j]], add=True)
        return 0

    lax.fori_loop(0, BPT, body, 0)
    plsc.subcore_barrier()
    pltpu.sync_copy(acc_sh.at[pl.ds(base, RPT)],
                    out_hbm.at[cid, pl.ds(base, RPT)])


_SC_PARAMS = pltpu.CompilerParams(use_tc_tiling_on_sc=False)

_deg_call = pl.kernel(
    _sc_degree_body,
    out_type=jax.ShapeDtypeStruct((NC, NPAD, LANES), jnp.float32),
    mesh=_MESH,
    compiler_params=_SC_PARAMS,
    scratch_types=[
        pltpu.VMEM((BPT, EPB), jnp.int32),
        pltpu.VMEM((EPB, LANES), jnp.float32),
        pltpu.VMEM((EPB, LANES), jnp.float32),
        pltpu.VMEM_SHARED((NPAD, LANES), jnp.float32),
    ],
)

_spmm_call = pl.kernel(
    _sc_spmm_body,
    out_type=jax.ShapeDtypeStruct((NC, NPAD, HID), jnp.float32),
    mesh=_MESH,
    compiler_params=_SC_PARAMS,
    scratch_types=[
        pltpu.VMEM((BPT, EPB), jnp.int32),
        pltpu.VMEM((BPT, EPB), jnp.int32),
        pltpu.VMEM((EPB, HID), jnp.float32),
        pltpu.VMEM((EPB, HID), jnp.float32),
        pltpu.VMEM_SHARED((NPAD, HID), jnp.float32),
    ],
)


_BM = 512
_GRID = NPAD // _BM


def _head_body(x_ref, w1_ref, degp_ref, dinv_ref, u1_ref):
    deg = degp_ref[0, :, 0:1] + degp_ref[1, :, 0:1] + 1.0
    dinv = lax.rsqrt(deg)
    dinv_ref[...] = dinv
    t = jnp.dot(x_ref[...], w1_ref[...], preferred_element_type=jnp.float32)
    u1_ref[...] = dinv * t


def _combine_body(n_mats, relu, pre_bias, post_scale, post_bias, *refs):
    i = 0
    acc_ref = refs[i]; i += 1
    u_ref = refs[i]; i += 1
    dinv_ref = refs[i]; i += 1
    b_ref = None
    if pre_bias:
        b_ref = refs[i]; i += 1
    w_refs = refs[i:i + n_mats]; i += n_mats
    pb_ref = None
    if post_bias:
        pb_ref = refs[i]; i += 1
    out_refs = refs[i:]

    dinv = dinv_ref[...]
    z = dinv * (acc_ref[0] + acc_ref[1] + u_ref[...])
    if pre_bias:
        z = z + b_ref[...]
    if relu:
        z = jnp.maximum(z, 0.0)
    if n_mats == 0:
        out_refs[0][...] = dinv * z if post_scale else z
    else:
        for w_ref, o_ref in zip(w_refs, out_refs):
            t = jnp.dot(z, w_ref[...], preferred_element_type=jnp.float32)
            if post_scale:
                t = dinv * t
            if post_bias:
                t = t + pb_ref[...]
            o_ref[...] = t


def _gram_body(a_ref, b_ref, o_ref):
    o_ref[...] = lax.dot_general(
        a_ref[...], b_ref[...], (((1,), (1,)), ((), ())),
        preferred_element_type=jnp.float32)


def _row_spec(width):
    return pl.BlockSpec((_BM, width), lambda i: (i, 0))


def _full_spec(r, c):
    return pl.BlockSpec((r, c), lambda i: (0, 0))


_ACC_SPEC = pl.BlockSpec((NC, _BM, HID), lambda i: (0, i, 0))

_head_call = pl.pallas_call(
    _head_body,
    grid=(_GRID,),
    in_specs=[_row_spec(IN_DIM), _full_spec(IN_DIM, HID),
              pl.BlockSpec((NC, _BM, LANES), lambda i: (0, i, 0))],
    out_specs=[_row_spec(1), _row_spec(HID)],
    out_shape=[jax.ShapeDtypeStruct((NPAD, 1), jnp.float32),
               jax.ShapeDtypeStruct((NPAD, HID), jnp.float32)],
)


def _make_combine(n_mats, relu, pre_bias, post_scale, post_bias,
                  out_widths, out_rows):
    in_specs = [_ACC_SPEC, _row_spec(HID), _row_spec(1)]
    if pre_bias:
        in_specs.append(_full_spec(1, HID))
    for _ in range(n_mats):
        in_specs.append(_full_spec(HID, out_widths[0]))
    if post_bias:
        in_specs.append(_full_spec(1, out_widths[0]))
    return pl.pallas_call(
        functools.partial(_combine_body, n_mats, relu, pre_bias,
                          post_scale, post_bias),
        grid=(_GRID,),
        in_specs=in_specs,
        out_specs=[_row_spec(w) for w in out_widths],
        out_shape=[jax.ShapeDtypeStruct((out_rows, w), jnp.float32)
                   for w in out_widths],
    )


_s1_call = _make_combine(1, True, True, True, False, [HID], NPAD)
_s2_call = _make_combine(2, False, True, True, False, [HID, HID], NPAD)
_s3_call = _make_combine(0, True, True, True, False, [HID], NPAD)
_s5_call = _make_combine(0, False, True, False, False, [HID], NPAD)
_s4_call = _make_combine(1, False, False, False, True, [IN_DIM], N_NODES)

_GB = 512
_gram_call = pl.pallas_call(
    _gram_body,
    grid=(pl.cdiv(N_NODES, _GB), pl.cdiv(N_NODES, _GB)),
    in_specs=[pl.BlockSpec((_GB, HID), lambda i, j: (i, 0)),
              pl.BlockSpec((_GB, HID), lambda i, j: (j, 0))],
    out_specs=pl.BlockSpec((_GB, _GB), lambda i, j: (i, j)),
    out_shape=jax.ShapeDtypeStruct((N_NODES, N_NODES), jnp.float32),
)


def kernel(x, edge_index, W1, b1, W2, b2, W3, b3, W4, b4, W5, b5):
    ei = edge_index.astype(jnp.int32)
    pad = jnp.full((ETOT - N_EDGES,), N_NODES, jnp.int32)
    srcp = jnp.concatenate([ei[0], pad]).reshape(NW * BPT, EPB)
    dstp = jnp.concatenate([ei[1], pad]).reshape(NW * BPT, EPB)

    b1r = b1.reshape(1, HID)
    b2r = b2.reshape(1, HID)
    b3r = b3.reshape(1, HID)
    b4r = b4.reshape(1, IN_DIM)
    b5r = b5.reshape(1, HID)

    degp = _deg_call(dstp)
    dinv, u1 = _head_call(x, W1, degp)

    acc1 = _spmm_call(u1, srcp, dstp)
    (u2,) = _s1_call(acc1, u1, dinv, b1r, W2)

    acc2 = _spmm_call(u2, srcp, dstp)
    u3, u5 = _s2_call(acc2, u2, dinv, b2r, W3, W5)

    acc5 = _spmm_call(u5, srcp, dstp)
    (h3,) = _s5_call(acc5, u5, dinv, b5r)

    acc3 = _spmm_call(u3, srcp, dstp)
    (u4,) = _s3_call(acc3, u3, dinv, b3r)

    acc4 = _spmm_call(u4, srcp, dstp)
    (x_,) = _s4_call(acc4, u4, dinv, W4, b4r)

    s_ = _gram_call(h3, h3)
    return (x_, s_)


# speedup vs baseline: 9.5372x; 1.1517x over previous
import functools

import jax
import jax.numpy as jnp
from jax import lax
from jax.experimental import pallas as pl
from jax.experimental.pallas import tpu as pltpu
from jax.experimental.pallas import tpu_sc as plsc

N_NODES = 10000
N_EDGES = 320000
IN_DIM = 128
HID = 64

NC, NS, LANES = 2, 16, 16
NW = NC * NS
EPB = 128
BPT = 80
EPT = EPB * BPT
ETOT = EPT * NW
NPAD = 10240
RPT = NPAD // NS

_MESH = plsc.VectorSubcoreMesh(core_axis_name="c", subcore_axis_name="s")


def _sc_degree_body(dst_hbm, out_hbm, idx_v, ones_v, zrow_v, acc_sh, sem):
    cid = lax.axis_index("c")
    sid = lax.axis_index("s")
    wid = cid * NS + sid
    base = sid * RPT

    def fill(i, _):
        zrow_v[i] = jnp.zeros((LANES,), jnp.float32)
        ones_v[i] = jnp.ones((LANES,), jnp.float32)
        return 0

    lax.fori_loop(0, EPB, fill, 0)
    for k in range(RPT // EPB):
        pltpu.sync_copy(zrow_v, acc_sh.at[pl.ds(base + k * EPB, EPB)])
    pltpu.sync_copy(dst_hbm.at[pl.ds(wid * BPT, BPT)], idx_v)
    plsc.subcore_barrier()

    def fire(j, _):
        pltpu.async_copy(ones_v, acc_sh.at[idx_v.at[j]], sem, add=True)
        return 0

    def drain(j, _):
        pltpu.make_async_copy(ones_v, acc_sh.at[idx_v.at[j]], sem).wait()
        return 0

    lax.fori_loop(0, BPT, fire, 0)
    lax.fori_loop(0, BPT, drain, 0)
    plsc.subcore_barrier()
    pltpu.sync_copy(acc_sh.at[pl.ds(base, RPT)],
                    out_hbm.at[cid, pl.ds(base, RPT)])


_NB = 2
_NT = BPT // _NB


def _sc_spmm_body(tbl_hbm, src_hbm, dst_hbm, out_hbm,
                  sidx_v, didx_v, rows_v, zrow_v, acc_sh, gsem, ssem, isem):
    cid = lax.axis_index("c")
    sid = lax.axis_index("s")
    wid = cid * NS + sid
    base = sid * RPT

    def g_src(j):
        return tbl_hbm.at[sidx_v.at[j]]

    def buf(slot):
        return rows_v.at[pl.ds(slot * EPB, EPB)]

    def s_dst(j):
        return acc_sh.at[didx_v.at[j]]

    pltpu.async_copy(src_hbm.at[pl.ds(wid * BPT, BPT)], sidx_v, isem.at[0])
    pltpu.async_copy(dst_hbm.at[pl.ds(wid * BPT, BPT)], didx_v, isem.at[1])

    def fill(i, _):
        for k in range(HID // LANES):
            zrow_v[i, pl.ds(k * LANES, LANES)] = jnp.zeros((LANES,), jnp.float32)
        return 0

    lax.fori_loop(0, EPB, fill, 0)
    for k in range(RPT // EPB):
        pltpu.sync_copy(zrow_v, acc_sh.at[pl.ds(base + k * EPB, EPB)])

    pltpu.make_async_copy(src_hbm.at[pl.ds(wid * BPT, BPT)], sidx_v,
                          isem.at[0]).wait()
    pltpu.make_async_copy(dst_hbm.at[pl.ds(wid * BPT, BPT)], didx_v,
                          isem.at[1]).wait()
    for b in range(_NB):
        pltpu.async_copy(g_src(b), buf(b), gsem.at[b])
    plsc.subcore_barrier()

    def body(t, _):
        s_cur = (t % 2) * _NB
        s_prev = _NB - s_cur
        for b in range(_NB):
            j = t * _NB + b
            pltpu.make_async_copy(g_src(j), buf(s_cur + b),
                                  gsem.at[s_cur + b]).wait()
            pltpu.async_copy(buf(s_cur + b), s_dst(j), ssem.at[s_cur + b],
                             add=True)
        for b in range(_NB):
            jp = (t - 1) * _NB + b

            @pl.when(t > 0)
            def _():
                pltpu.make_async_copy(buf(s_prev + b), s_dst(jp),
                                      ssem.at[s_prev + b]).wait()

            jn = (t + 1) * _NB + b

            @pl.when(t < _NT - 1)
            def _():
                pltpu.async_copy(g_src(jn), buf(s_prev + b),
                                 gsem.at[s_prev + b])
        return 0

    lax.fori_loop(0, _NT, body, 0)
    s_last = ((_NT - 1) % 2) * _NB
    for b in range(_NB):
        j = (_NT - 1) * _NB + b
        pltpu.make_async_copy(buf(s_last + b), s_dst(j),
                              ssem.at[s_last + b]).wait()
    plsc.subcore_barrier()
    pltpu.sync_copy(acc_sh.at[pl.ds(base, RPT)],
                    out_hbm.at[cid, pl.ds(base, RPT)])


_SC_PARAMS = pltpu.CompilerParams(use_tc_tiling_on_sc=False)

_deg_call = pl.kernel(
    _sc_degree_body,
    out_type=jax.ShapeDtypeStruct((NC, NPAD, LANES), jnp.float32),
    mesh=_MESH,
    compiler_params=_SC_PARAMS,
    scratch_types=[
        pltpu.VMEM((BPT, EPB), jnp.int32),
        pltpu.VMEM((EPB, LANES), jnp.float32),
        pltpu.VMEM((EPB, LANES), jnp.float32),
        pltpu.VMEM_SHARED((NPAD, LANES), jnp.float32),
        pltpu.SemaphoreType.DMA,
    ],
)

_spmm_call = pl.kernel(
    _sc_spmm_body,
    out_type=jax.ShapeDtypeStruct((NC, NPAD, HID), jnp.float32),
    mesh=_MESH,
    compiler_params=_SC_PARAMS,
    scratch_types=[
        pltpu.VMEM((BPT, EPB), jnp.int32),
        pltpu.VMEM((BPT, EPB), jnp.int32),
        pltpu.VMEM((2 * _NB * EPB, HID), jnp.float32),
        pltpu.VMEM((EPB, HID), jnp.float32),
        pltpu.VMEM_SHARED((NPAD, HID), jnp.float32),
        pltpu.SemaphoreType.DMA((2 * _NB,)),
        pltpu.SemaphoreType.DMA((2 * _NB,)),
        pltpu.SemaphoreType.DMA((2,)),
    ],
)


_BM = 512
_GRID = NPAD // _BM


def _head_body(x_ref, w1_ref, degp_ref, dinv_ref, u1_ref):
    deg = degp_ref[0, :, 0:1] + degp_ref[1, :, 0:1] + 1.0
    dinv = lax.rsqrt(deg)
    dinv_ref[...] = dinv
    t = jnp.dot(x_ref[...], w1_ref[...], preferred_element_type=jnp.float32)
    u1_ref[...] = dinv * t


def _combine_body(n_mats, relu, pre_bias, post_scale, post_bias, *refs):
    i = 0
    acc_ref = refs[i]; i += 1
    u_ref = refs[i]; i += 1
    dinv_ref = refs[i]; i += 1
    b_ref = None
    if pre_bias:
        b_ref = refs[i]; i += 1
    w_refs = refs[i:i + n_mats]; i += n_mats
    pb_ref = None
    if post_bias:
        pb_ref = refs[i]; i += 1
    out_refs = refs[i:]

    dinv = dinv_ref[...]
    z = dinv * (acc_ref[0] + acc_ref[1] + u_ref[...])
    if pre_bias:
        z = z + b_ref[...]
    if relu:
        z = jnp.maximum(z, 0.0)
    if n_mats == 0:
        out_refs[0][...] = dinv * z if post_scale else z
    else:
        for w_ref, o_ref in zip(w_refs, out_refs):
            t = jnp.dot(z, w_ref[...], preferred_element_type=jnp.float32)
            if post_scale:
                t = dinv * t
            if post_bias:
                t = t + pb_ref[...]
            o_ref[...] = t


def _gram_body(a_ref, b_ref, o_ref):
    o_ref[...] = lax.dot_general(
        a_ref[...], b_ref[...], (((1,), (1,)), ((), ())),
        preferred_element_type=jnp.float32)


def _row_spec(width):
    return pl.BlockSpec((_BM, width), lambda i: (i, 0))


def _full_spec(r, c):
    return pl.BlockSpec((r, c), lambda i: (0, 0))


_ACC_SPEC = pl.BlockSpec((NC, _BM, HID), lambda i: (0, i, 0))

_head_call = pl.pallas_call(
    _head_body,
    grid=(_GRID,),
    in_specs=[_row_spec(IN_DIM), _full_spec(IN_DIM, HID),
              pl.BlockSpec((NC, _BM, LANES), lambda i: (0, i, 0))],
    out_specs=[_row_spec(1), _row_spec(HID)],
    out_shape=[jax.ShapeDtypeStruct((NPAD, 1), jnp.float32),
               jax.ShapeDtypeStruct((NPAD, HID), jnp.float32)],
)


def _make_combine(n_mats, relu, pre_bias, post_scale, post_bias,
                  out_widths, out_rows):
    in_specs = [_ACC_SPEC, _row_spec(HID), _row_spec(1)]
    if pre_bias:
        in_specs.append(_full_spec(1, HID))
    for _ in range(n_mats):
        in_specs.append(_full_spec(HID, out_widths[0]))
    if post_bias:
        in_specs.append(_full_spec(1, out_widths[0]))
    return pl.pallas_call(
        functools.partial(_combine_body, n_mats, relu, pre_bias,
                          post_scale, post_bias),
        grid=(_GRID,),
        in_specs=in_specs,
        out_specs=[_row_spec(w) for w in out_widths],
        out_shape=[jax.ShapeDtypeStruct((out_rows, w), jnp.float32)
                   for w in out_widths],
    )


_s1_call = _make_combine(1, True, True, True, False, [HID], NPAD)
_s2_call = _make_combine(2, False, True, True, False, [HID, HID], NPAD)
_s3_call = _make_combine(0, True, True, True, False, [HID], NPAD)
_s5_call = _make_combine(0, False, True, False, False, [HID], NPAD)
_s4_call = _make_combine(1, False, False, False, True, [IN_DIM], N_NODES)

_GB = 512
_gram_call = pl.pallas_call(
    _gram_body,
    grid=(pl.cdiv(N_NODES, _GB), pl.cdiv(N_NODES, _GB)),
    in_specs=[pl.BlockSpec((_GB, HID), lambda i, j: (i, 0)),
              pl.BlockSpec((_GB, HID), lambda i, j: (j, 0))],
    out_specs=pl.BlockSpec((_GB, _GB), lambda i, j: (i, j)),
    out_shape=jax.ShapeDtypeStruct((N_NODES, N_NODES), jnp.float32),
)


def kernel(x, edge_index, W1, b1, W2, b2, W3, b3, W4, b4, W5, b5):
    ei = edge_index.astype(jnp.int32)
    pad = jnp.full((ETOT - N_EDGES,), N_NODES, jnp.int32)
    srcp = jnp.concatenate([ei[0], pad]).reshape(NW * BPT, EPB)
    dstp = jnp.concatenate([ei[1], pad]).reshape(NW * BPT, EPB)

    b1r = b1.reshape(1, HID)
    b2r = b2.reshape(1, HID)
    b3r = b3.reshape(1, HID)
    b4r = b4.reshape(1, IN_DIM)
    b5r = b5.reshape(1, HID)

    degp = _deg_call(dstp)
    dinv, u1 = _head_call(x, W1, degp)

    acc1 = _spmm_call(u1, srcp, dstp)
    (u2,) = _s1_call(acc1, u1, dinv, b1r, W2)

    acc2 = _spmm_call(u2, srcp, dstp)
    u3, u5 = _s2_call(acc2, u2, dinv, b2r, W3, W5)

    acc5 = _spmm_call(u5, srcp, dstp)
    (h3,) = _s5_call(acc5, u5, dinv, b5r)

    u3, _ = lax.optimization_barrier((u3, acc5))
    acc3 = _spmm_call(u3, srcp, dstp)
    (u4,) = _s3_call(acc3, u3, dinv, b3r)

    acc4 = _spmm_call(u4, srcp, dstp)
    (x_,) = _s4_call(acc4, u4, dinv, W4, b4r)

    s_ = _gram_call(h3, h3)
    return (x_, s_)


# speedup vs baseline: 17.3415x; 1.8183x over previous
import functools

import jax
import jax.numpy as jnp
from jax import lax
from jax.experimental import pallas as pl
from jax.experimental.pallas import tpu as pltpu
from jax.experimental.pallas import tpu_sc as plsc

N_NODES = 10000
N_EDGES = 320000
IN_DIM = 128
HID = 64

NC, NS, LANES = 2, 16, 16
NW = NC * NS
EPB = 128
BPT = 80
EPT = EPB * BPT
ETOT = EPT * NW
NPAD = 10240
RPT = NPAD // NS

_MESH = plsc.VectorSubcoreMesh(core_axis_name="c", subcore_axis_name="s")


def _sc_degree_body(dst_hbm, out_hbm, idx_v, ones_v, zrow_v, acc_sh, sem):
    cid = lax.axis_index("c")
    sid = lax.axis_index("s")
    wid = cid * NS + sid
    base = sid * RPT

    def fill(i, _):
        zrow_v[i] = jnp.zeros((LANES,), jnp.float32)
        ones_v[i] = jnp.ones((LANES,), jnp.float32)
        return 0

    lax.fori_loop(0, EPB, fill, 0)
    for k in range(RPT // EPB):
        pltpu.sync_copy(zrow_v, acc_sh.at[pl.ds(base + k * EPB, EPB)])
    pltpu.sync_copy(dst_hbm.at[pl.ds(wid * BPT, BPT)], idx_v)
    plsc.subcore_barrier()

    def fire(j, _):
        pltpu.async_copy(ones_v, acc_sh.at[idx_v.at[j]], sem, add=True)
        return 0

    def drain(j, _):
        pltpu.make_async_copy(ones_v, acc_sh.at[idx_v.at[j]], sem).wait()
        return 0

    lax.fori_loop(0, BPT, fire, 0)
    lax.fori_loop(0, BPT, drain, 0)
    plsc.subcore_barrier()
    pltpu.sync_copy(acc_sh.at[pl.ds(base, RPT)],
                    out_hbm.at[cid, pl.ds(base, RPT)])


_NJ = BPT


def _sc_spmm_body(tbl_hbm, src_hbm, dst_hbm, out_hbm,
                  sidx_v, didx_v, rows_v, acc_sh, tbl_sh,
                  gsem, ssem, isem):
    cid = lax.axis_index("c")
    sid = lax.axis_index("s")
    wid = cid * NS + sid
    base = sid * RPT

    def g_src(j):
        return tbl_sh.at[sidx_v.at[j]]

    def buf(slot):
        return rows_v.at[pl.ds(slot * EPB, EPB)]

    def s_dst(j):
        return acc_sh.at[didx_v.at[j]]

    pltpu.async_copy(src_hbm.at[pl.ds(wid * BPT, BPT)], sidx_v, isem.at[0])
    pltpu.async_copy(dst_hbm.at[pl.ds(wid * BPT, BPT)], didx_v, isem.at[1])

    def fill(i, _):
        for k in range(HID // LANES):
            rows_v[i, pl.ds(k * LANES, LANES)] = jnp.zeros((LANES,), jnp.float32)
        return 0

    lax.fori_loop(0, EPB, fill, 0)
    pltpu.sync_copy(tbl_hbm.at[pl.ds(base, RPT)], tbl_sh.at[pl.ds(base, RPT)])
    for k in range(RPT // EPB):
        pltpu.sync_copy(buf(0), acc_sh.at[pl.ds(base + k * EPB, EPB)])

    pltpu.make_async_copy(src_hbm.at[pl.ds(wid * BPT, BPT)], sidx_v,
                          isem.at[0]).wait()
    pltpu.make_async_copy(dst_hbm.at[pl.ds(wid * BPT, BPT)], didx_v,
                          isem.at[1]).wait()
    plsc.subcore_barrier()
    pltpu.async_copy(g_src(0), buf(0), gsem.at[0])
    pltpu.async_copy(g_src(1), buf(1), gsem.at[1])

    def body(t, _):
        j0 = 2 * t
        pltpu.make_async_copy(g_src(j0), buf(0), gsem.at[0]).wait()
        pltpu.async_copy(buf(0), s_dst(j0), ssem.at[0], add=True)

        @pl.when(t > 0)
        def _():
            pltpu.make_async_copy(buf(1), s_dst(j0 - 1), ssem.at[1]).wait()
            pltpu.async_copy(g_src(j0 + 1), buf(1), gsem.at[1])

        j1 = 2 * t + 1
        pltpu.make_async_copy(g_src(j1), buf(1), gsem.at[1]).wait()
        pltpu.async_copy(buf(1), s_dst(j1), ssem.at[1], add=True)
        pltpu.make_async_copy(buf(0), s_dst(j0), ssem.at[0]).wait()

        @pl.when(t < _NJ // 2 - 1)
        def _():
            pltpu.async_copy(g_src(j1 + 1), buf(0), gsem.at[0])
        return 0

    lax.fori_loop(0, _NJ // 2, body, 0)
    pltpu.make_async_copy(buf(1), s_dst(_NJ - 1), ssem.at[1]).wait()
    plsc.subcore_barrier()
    pltpu.sync_copy(acc_sh.at[pl.ds(base, RPT)],
                    out_hbm.at[cid, pl.ds(base, RPT)])


_SC_PARAMS = pltpu.CompilerParams(use_tc_tiling_on_sc=False)

_deg_call = pl.kernel(
    _sc_degree_body,
    out_type=jax.ShapeDtypeStruct((NC, NPAD, LANES), jnp.float32),
    mesh=_MESH,
    compiler_params=_SC_PARAMS,
    scratch_types=[
        pltpu.VMEM((BPT, EPB), jnp.int32),
        pltpu.VMEM((EPB, LANES), jnp.float32),
        pltpu.VMEM((EPB, LANES), jnp.float32),
        pltpu.VMEM_SHARED((NPAD, LANES), jnp.float32),
        pltpu.SemaphoreType.DMA,
    ],
)

_spmm_call = pl.kernel(
    _sc_spmm_body,
    out_type=jax.ShapeDtypeStruct((NC, NPAD, HID), jnp.float32),
    mesh=_MESH,
    compiler_params=_SC_PARAMS,
    scratch_types=[
        pltpu.VMEM((BPT, EPB), jnp.int32),
        pltpu.VMEM((BPT, EPB), jnp.int32),
        pltpu.VMEM((2 * EPB, HID), jnp.float32),
        pltpu.VMEM_SHARED((NPAD, HID), jnp.float32),
        pltpu.VMEM_SHARED((NPAD, HID), jnp.float32),
        pltpu.SemaphoreType.DMA((2,)),
        pltpu.SemaphoreType.DMA((2,)),
        pltpu.SemaphoreType.DMA((2,)),
    ],
)


_BM = 512
_GRID = NPAD // _BM


def _head_body(x_ref, w1_ref, degp_ref, dinv_ref, u1_ref):
    deg = degp_ref[0, :, 0:1] + degp_ref[1, :, 0:1] + 1.0
    dinv = lax.rsqrt(deg)
    dinv_ref[...] = dinv
    t = jnp.dot(x_ref[...], w1_ref[...], preferred_element_type=jnp.float32)
    u1_ref[...] = dinv * t


def _combine_body(n_mats, relu, pre_bias, post_scale, post_bias, *refs):
    i = 0
    acc_ref = refs[i]; i += 1
    u_ref = refs[i]; i += 1
    dinv_ref = refs[i]; i += 1
    b_ref = None
    if pre_bias:
        b_ref = refs[i]; i += 1
    w_refs = refs[i:i + n_mats]; i += n_mats
    pb_ref = None
    if post_bias:
        pb_ref = refs[i]; i += 1
    out_refs = refs[i:]

    dinv = dinv_ref[...]
    z = dinv * (acc_ref[0] + acc_ref[1] + u_ref[...])
    if pre_bias:
        z = z + b_ref[...]
    if relu:
        z = jnp.maximum(z, 0.0)
    if n_mats == 0:
        out_refs[0][...] = dinv * z if post_scale else z
    else:
        for w_ref, o_ref in zip(w_refs, out_refs):
            t = jnp.dot(z, w_ref[...], preferred_element_type=jnp.float32)
            if post_scale:
                t = dinv * t
            if post_bias:
                t = t + pb_ref[...]
            o_ref[...] = t


def _gram_body(a_ref, b_ref, o_ref):
    o_ref[...] = lax.dot_general(
        a_ref[...], b_ref[...], (((1,), (1,)), ((), ())),
        preferred_element_type=jnp.float32)


def _row_spec(width):
    return pl.BlockSpec((_BM, width), lambda i: (i, 0))


def _full_spec(r, c):
    return pl.BlockSpec((r, c), lambda i: (0, 0))


_ACC_SPEC = pl.BlockSpec((NC, _BM, HID), lambda i: (0, i, 0))

_head_call = pl.pallas_call(
    _head_body,
    grid=(_GRID,),
    in_specs=[_row_spec(IN_DIM), _full_spec(IN_DIM, HID),
              pl.BlockSpec((NC, _BM, LANES), lambda i: (0, i, 0))],
    out_specs=[_row_spec(1), _row_spec(HID)],
    out_shape=[jax.ShapeDtypeStruct((NPAD, 1), jnp.float32),
               jax.ShapeDtypeStruct((NPAD, HID), jnp.float32)],
)


def _make_combine(n_mats, relu, pre_bias, post_scale, post_bias,
                  out_widths, out_rows):
    in_specs = [_ACC_SPEC, _row_spec(HID), _row_spec(1)]
    if pre_bias:
        in_specs.append(_full_spec(1, HID))
    for _ in range(n_mats):
        in_specs.append(_full_spec(HID, out_widths[0]))
    if post_bias:
        in_specs.append(_full_spec(1, out_widths[0]))
    return pl.pallas_call(
        functools.partial(_combine_body, n_mats, relu, pre_bias,
                          post_scale, post_bias),
        grid=(_GRID,),
        in_specs=in_specs,
        out_specs=[_row_spec(w) for w in out_widths],
        out_shape=[jax.ShapeDtypeStruct((out_rows, w), jnp.float32)
                   for w in out_widths],
    )


_s1_call = _make_combine(1, True, True, True, False, [HID], NPAD)
_s2_call = _make_combine(2, False, True, True, False, [HID, HID], NPAD)
_s3_call = _make_combine(0, True, True, True, False, [HID], NPAD)
_s5_call = _make_combine(0, False, True, False, False, [HID], NPAD)
_s4_call = _make_combine(1, False, False, False, True, [IN_DIM], N_NODES)

_GB = 512
_gram_call = pl.pallas_call(
    _gram_body,
    grid=(pl.cdiv(N_NODES, _GB), pl.cdiv(N_NODES, _GB)),
    in_specs=[pl.BlockSpec((_GB, HID), lambda i, j: (i, 0)),
              pl.BlockSpec((_GB, HID), lambda i, j: (j, 0))],
    out_specs=pl.BlockSpec((_GB, _GB), lambda i, j: (i, j)),
    out_shape=jax.ShapeDtypeStruct((N_NODES, N_NODES), jnp.float32),
)


def kernel(x, edge_index, W1, b1, W2, b2, W3, b3, W4, b4, W5, b5):
    ei = edge_index.astype(jnp.int32)
    pad = jnp.full((ETOT - N_EDGES,), N_NODES, jnp.int32)
    srcp = jnp.concatenate([ei[0], pad]).reshape(NW * BPT, EPB)
    dstp = jnp.concatenate([ei[1], pad]).reshape(NW * BPT, EPB)

    b1r = b1.reshape(1, HID)
    b2r = b2.reshape(1, HID)
    b3r = b3.reshape(1, HID)
    b4r = b4.reshape(1, IN_DIM)
    b5r = b5.reshape(1, HID)

    degp = _deg_call(dstp)
    dinv, u1 = _head_call(x, W1, degp)

    acc1 = _spmm_call(u1, srcp, dstp)
    (u2,) = _s1_call(acc1, u1, dinv, b1r, W2)

    acc2 = _spmm_call(u2, srcp, dstp)
    u3, u5 = _s2_call(acc2, u2, dinv, b2r, W3, W5)

    acc5 = _spmm_call(u5, srcp, dstp)
    (h3,) = _s5_call(acc5, u5, dinv, b5r)

    u3, _ = lax.optimization_barrier((u3, acc5))
    acc3 = _spmm_call(u3, srcp, dstp)
    (u4,) = _s3_call(acc3, u3, dinv, b3r)

    acc4 = _spmm_call(u4, srcp, dstp)
    (x_,) = _s4_call(acc4, u4, dinv, W4, b4r)

    s_ = _gram_call(h3, h3)
    return (x_, s_)


# speedup vs baseline: 23.2462x; 1.3405x over previous
import functools

import jax
import jax.numpy as jnp
from jax import lax
from jax.experimental import pallas as pl
from jax.experimental.pallas import tpu as pltpu
from jax.experimental.pallas import tpu_sc as plsc

N_NODES = 10000
N_EDGES = 320000
IN_DIM = 128
HID = 64

NC, NS, LANES = 2, 16, 16
NW = NC * NS
EPB = 128
BPT = 80
EPT = EPB * BPT
ETOT = EPT * NW
NPAD = 10240
RPT = NPAD // NS

_MESH = plsc.VectorSubcoreMesh(core_axis_name="c", subcore_axis_name="s")


def _sc_degree_body(dst_hbm, out_hbm, idx_v, ones_v, zrow_v, acc_sh, sem):
    cid = lax.axis_index("c")
    sid = lax.axis_index("s")
    wid = cid * NS + sid
    base = sid * RPT

    def fill(i, _):
        zrow_v[i] = jnp.zeros((LANES,), jnp.float32)
        ones_v[i] = jnp.ones((LANES,), jnp.float32)
        return 0

    lax.fori_loop(0, EPB, fill, 0)
    for k in range(RPT // EPB):
        pltpu.sync_copy(zrow_v, acc_sh.at[pl.ds(base + k * EPB, EPB)])
    pltpu.sync_copy(dst_hbm.at[pl.ds(wid * BPT, BPT)], idx_v)
    plsc.subcore_barrier()

    def fire(j, _):
        pltpu.async_copy(ones_v, acc_sh.at[idx_v.at[j]], sem, add=True)
        return 0

    def drain(j, _):
        pltpu.make_async_copy(ones_v, acc_sh.at[idx_v.at[j]], sem).wait()
        return 0

    lax.fori_loop(0, BPT, fire, 0)
    lax.fori_loop(0, BPT, drain, 0)
    plsc.subcore_barrier()
    pltpu.sync_copy(acc_sh.at[pl.ds(base, RPT)],
                    out_hbm.at[cid, pl.ds(base, RPT)])


_NJ = BPT


def _sc_spmm_body(tbl_hbm, src_hbm, dst_hbm, out_hbm,
                  sidx_v, didx_v, rows_v, acc_sh, tbl_sh,
                  gsem, ssem, isem):
    cid = lax.axis_index("c")
    sid = lax.axis_index("s")
    wid = cid * NS + sid
    base = sid * RPT

    def g_src(j):
        return tbl_sh.at[sidx_v.at[j]]

    def buf(slot):
        return rows_v.at[pl.ds(slot * EPB, EPB)]

    def s_dst(j):
        return acc_sh.at[didx_v.at[j]]

    pltpu.async_copy(src_hbm.at[pl.ds(wid * BPT, BPT)], sidx_v, isem.at[0])
    pltpu.async_copy(dst_hbm.at[pl.ds(wid * BPT, BPT)], didx_v, isem.at[1])

    def fill(i, _):
        for k in range(HID // LANES):
            rows_v[i, pl.ds(k * LANES, LANES)] = jnp.zeros((LANES,), jnp.float32)
        return 0

    lax.fori_loop(0, EPB, fill, 0)
    pltpu.sync_copy(tbl_hbm.at[pl.ds(base, RPT)], tbl_sh.at[pl.ds(base, RPT)])
    for k in range(RPT // EPB):
        pltpu.sync_copy(buf(0), acc_sh.at[pl.ds(base + k * EPB, EPB)])

    pltpu.make_async_copy(src_hbm.at[pl.ds(wid * BPT, BPT)], sidx_v,
                          isem.at[0]).wait()
    pltpu.make_async_copy(dst_hbm.at[pl.ds(wid * BPT, BPT)], didx_v,
                          isem.at[1]).wait()
    plsc.subcore_barrier()
    pltpu.async_copy(g_src(0), buf(0), gsem.at[0])
    pltpu.async_copy(g_src(1), buf(1), gsem.at[1])

    def body(t, _):
        j0 = 2 * t
        pltpu.make_async_copy(g_src(j0), buf(0), gsem.at[0]).wait()
        pltpu.async_copy(buf(0), s_dst(j0), ssem.at[0], add=True)

        @pl.when(t > 0)
        def _():
            pltpu.make_async_copy(buf(1), s_dst(j0 - 1), ssem.at[1]).wait()
            pltpu.async_copy(g_src(j0 + 1), buf(1), gsem.at[1])

        j1 = 2 * t + 1
        pltpu.make_async_copy(g_src(j1), buf(1), gsem.at[1]).wait()
        pltpu.async_copy(buf(1), s_dst(j1), ssem.at[1], add=True)
        pltpu.make_async_copy(buf(0), s_dst(j0), ssem.at[0]).wait()

        @pl.when(t < _NJ // 2 - 1)
        def _():
            pltpu.async_copy(g_src(j1 + 1), buf(0), gsem.at[0])
        return 0

    lax.fori_loop(0, _NJ // 2, body, 0)
    pltpu.make_async_copy(buf(1), s_dst(_NJ - 1), ssem.at[1]).wait()
    plsc.subcore_barrier()
    pltpu.sync_copy(acc_sh.at[pl.ds(base, RPT)],
                    out_hbm.at[cid, pl.ds(base, RPT)])


_SC_PARAMS = pltpu.CompilerParams(use_tc_tiling_on_sc=False)

_deg_call = pl.kernel(
    _sc_degree_body,
    out_type=jax.ShapeDtypeStruct((NC, NPAD, LANES), jnp.float32),
    mesh=_MESH,
    compiler_params=_SC_PARAMS,
    scratch_types=[
        pltpu.VMEM((BPT, EPB), jnp.int32),
        pltpu.VMEM((EPB, LANES), jnp.float32),
        pltpu.VMEM((EPB, LANES), jnp.float32),
        pltpu.VMEM_SHARED((NPAD, LANES), jnp.float32),
        pltpu.SemaphoreType.DMA,
    ],
)

_spmm_call = pl.kernel(
    _sc_spmm_body,
    out_type=jax.ShapeDtypeStruct((NC, NPAD, HID), jnp.float32),
    mesh=_MESH,
    compiler_params=_SC_PARAMS,
    scratch_types=[
        pltpu.VMEM((BPT, EPB), jnp.int32),
        pltpu.VMEM((BPT, EPB), jnp.int32),
        pltpu.VMEM((2 * EPB, HID), jnp.float32),
        pltpu.VMEM_SHARED((NPAD, HID), jnp.float32),
        pltpu.VMEM_SHARED((NPAD, HID), jnp.float32),
        pltpu.SemaphoreType.DMA((2,)),
        pltpu.SemaphoreType.DMA((2,)),
        pltpu.SemaphoreType.DMA((2,)),
    ],
)

_BM = 512
_GRID = NPAD // _BM


def _head_body(x_ref, w1_ref, degp_ref, dinv_ref, u1_ref):
    deg = degp_ref[0, :, 0:1] + degp_ref[1, :, 0:1] + 1.0
    dinv = lax.rsqrt(deg)
    dinv_ref[...] = dinv
    t = jnp.dot(x_ref[...], w1_ref[...], preferred_element_type=jnp.float32)
    u1_ref[...] = dinv * t


def _combine_body(n_mats, relu, pre_bias, post_scale, post_bias, *refs):
    i = 0
    acc_ref = refs[i]; i += 1
    u_ref = refs[i]; i += 1
    dinv_ref = refs[i]; i += 1
    b_ref = None
    if pre_bias:
        b_ref = refs[i]; i += 1
    w_refs = refs[i:i + n_mats]; i += n_mats
    pb_ref = None
    if post_bias:
        pb_ref = refs[i]; i += 1
    out_refs = refs[i:]

    dinv = dinv_ref[...]
    z = dinv * (acc_ref[0] + acc_ref[1] + u_ref[...])
    if pre_bias:
        z = z + b_ref[...]
    if relu:
        z = jnp.maximum(z, 0.0)
    if n_mats == 0:
        out_refs[0][...] = dinv * z if post_scale else z
    else:
        for w_ref, o_ref in zip(w_refs, out_refs):
            t = jnp.dot(z, w_ref[...], preferred_element_type=jnp.float32)
            if post_scale:
                t = dinv * t
            if post_bias:
                t = t + pb_ref[...]
            o_ref[...] = t


def _gram_body(a_ref, b_ref, o_ref):
    res = lax.dot_general(
        a_ref[...], b_ref[...], (((1,), (1,)), ((), ())),
        preferred_element_type=jnp.float32)
    o_ref[...] = res[:, :N_NODES]


def _row_spec(width):
    return pl.BlockSpec((_BM, width), lambda i: (i, 0))


def _full_spec(r, c):
    return pl.BlockSpec((r, c), lambda i: (0, 0))


_ACC_SPEC = pl.BlockSpec((NC, _BM, HID), lambda i: (0, i, 0))

_head_call = pl.pallas_call(
    _head_body,
    grid=(_GRID,),
    in_specs=[_row_spec(IN_DIM), _full_spec(IN_DIM, HID),
              pl.BlockSpec((NC, _BM, LANES), lambda i: (0, i, 0))],
    out_specs=[_row_spec(1), _row_spec(HID)],
    out_shape=[jax.ShapeDtypeStruct((NPAD, 1), jnp.float32),
               jax.ShapeDtypeStruct((NPAD, HID), jnp.float32)],
)


def _make_combine(n_mats, relu, pre_bias, post_scale, post_bias,
                  out_widths, out_rows):
    in_specs = [_ACC_SPEC, _row_spec(HID), _row_spec(1)]
    if pre_bias:
        in_specs.append(_full_spec(1, HID))
    for _ in range(n_mats):
        in_specs.append(_full_spec(HID, out_widths[0]))
    if post_bias:
        in_specs.append(_full_spec(1, out_widths[0]))
    return pl.pallas_call(
        functools.partial(_combine_body, n_mats, relu, pre_bias,
                          post_scale, post_bias),
        grid=(_GRID,),
        in_specs=in_specs,
        out_specs=[_row_spec(w) for w in out_widths],
        out_shape=[jax.ShapeDtypeStruct((out_rows, w), jnp.float32)
                   for w in out_widths],
    )


_s1_call = _make_combine(1, True, True, True, False, [HID], NPAD)
_s2_call = _make_combine(2, False, True, True, False, [HID, HID], NPAD)
_s3_call = _make_combine(0, True, True, True, False, [HID], NPAD)
_s5_call = _make_combine(0, False, True, False, False, [HID], NPAD)
_s4_call = _make_combine(1, False, False, False, True, [IN_DIM], N_NODES)

_GB = 256
_gram_call = pl.pallas_call(
    _gram_body,
    grid=(pl.cdiv(N_NODES, _GB),),
    in_specs=[pl.BlockSpec((_GB, HID), lambda i: (i, 0)),
              pl.BlockSpec((NPAD, HID), lambda i: (0, 0))],
    out_specs=pl.BlockSpec((_GB, N_NODES), lambda i: (i, 0)),
    out_shape=jax.ShapeDtypeStruct((N_NODES, N_NODES), jnp.float32),
)


def kernel(x, edge_index, W1, b1, W2, b2, W3, b3, W4, b4, W5, b5):
    ei = edge_index.astype(jnp.int32)
    pad = jnp.full((ETOT - N_EDGES,), N_NODES, jnp.int32)
    srcp = jnp.concatenate([ei[0], pad]).reshape(NW * BPT, EPB)
    dstp = jnp.concatenate([ei[1], pad]).reshape(NW * BPT, EPB)

    b1r = b1.reshape(1, HID)
    b2r = b2.reshape(1, HID)
    b3r = b3.reshape(1, HID)
    b4r = b4.reshape(1, IN_DIM)
    b5r = b5.reshape(1, HID)

    degp = _deg_call(dstp)
    dinv, u1 = _head_call(x, W1, degp)

    acc1 = _spmm_call(u1, srcp, dstp)
    (u2,) = _s1_call(acc1, u1, dinv, b1r, W2)

    acc2 = _spmm_call(u2, srcp, dstp)
    u3, u5 = _s2_call(acc2, u2, dinv, b2r, W3, W5)

    acc5 = _spmm_call(u5, srcp, dstp)
    (h3,) = _s5_call(acc5, u5, dinv, b5r)

    u3, _ = lax.optimization_barrier((u3, acc5))
    acc3 = _spmm_call(u3, srcp, dstp)
    (u4,) = _s3_call(acc3, u3, dinv, b3r)

    acc4 = _spmm_call(u4, srcp, dstp)
    (x_,) = _s4_call(acc4, u4, dinv, W4, b4r)

    s_ = _gram_call(h3, h3)
    return (x_, s_)


# speedup vs baseline: 27.6149x; 1.1879x over previous
import functools

import jax
import jax.numpy as jnp
from jax import lax
from jax.experimental import pallas as pl
from jax.experimental.pallas import tpu as pltpu
from jax.experimental.pallas import tpu_sc as plsc

N_NODES = 10000
N_EDGES = 320000
IN_DIM = 128
HID = 64

NC, NS, LANES = 2, 16, 16
NW = NC * NS
EPB = 128
BPT = 80
EPT = EPB * BPT
ETOT = EPT * NW
NPAD = 10240
RPT = NPAD // NS

_MESH = plsc.VectorSubcoreMesh(core_axis_name="c", subcore_axis_name="s")


def _sc_degree_body(dst_hbm, out_hbm, idx_v, ones_v, zrow_v, acc_sh, sem):
    cid = lax.axis_index("c")
    sid = lax.axis_index("s")
    wid = cid * NS + sid
    base = sid * RPT

    def fill(i, _):
        zrow_v[i] = jnp.zeros((LANES,), jnp.float32)
        ones_v[i] = jnp.ones((LANES,), jnp.float32)
        return 0

    lax.fori_loop(0, EPB, fill, 0)
    for k in range(RPT // EPB):
        pltpu.sync_copy(zrow_v, acc_sh.at[pl.ds(base + k * EPB, EPB)])
    pltpu.sync_copy(dst_hbm.at[pl.ds(wid * BPT, BPT)], idx_v)
    plsc.subcore_barrier()

    def fire(j, _):
        pltpu.async_copy(ones_v, acc_sh.at[idx_v.at[j]], sem, add=True)
        return 0

    def drain(j, _):
        pltpu.make_async_copy(ones_v, acc_sh.at[idx_v.at[j]], sem).wait()
        return 0

    lax.fori_loop(0, BPT, fire, 0)
    lax.fori_loop(0, BPT, drain, 0)
    plsc.subcore_barrier()
    pltpu.sync_copy(acc_sh.at[pl.ds(base, RPT)],
                    out_hbm.at[cid, pl.ds(base, RPT)])


_NJ = BPT


def _sc_spmm_body(tbl_hbm, src_hbm, dst_hbm, out_hbm,
                  sidx_v, didx_v, rows_v, acc_sh, tbl_sh,
                  gsem, ssem, isem):
    cid = lax.axis_index("c")
    sid = lax.axis_index("s")
    wid = cid * NS + sid
    base = sid * RPT

    def g_src(j):
        return tbl_sh.at[sidx_v.at[j]]

    def buf(slot):
        return rows_v.at[pl.ds(slot * EPB, EPB)]

    def s_dst(j):
        return acc_sh.at[didx_v.at[j]]

    pltpu.async_copy(src_hbm.at[pl.ds(wid * BPT, BPT)], sidx_v, isem.at[0])
    pltpu.async_copy(dst_hbm.at[pl.ds(wid * BPT, BPT)], didx_v, isem.at[1])

    def fill(i, _):
        for k in range(HID // LANES):
            rows_v[i, pl.ds(k * LANES, LANES)] = jnp.zeros((LANES,), jnp.float32)
        return 0

    lax.fori_loop(0, EPB, fill, 0)
    pltpu.sync_copy(tbl_hbm.at[pl.ds(base, RPT)], tbl_sh.at[pl.ds(base, RPT)])
    for k in range(RPT // EPB):
        pltpu.sync_copy(buf(0), acc_sh.at[pl.ds(base + k * EPB, EPB)])

    pltpu.make_async_copy(src_hbm.at[pl.ds(wid * BPT, BPT)], sidx_v,
                          isem.at[0]).wait()
    pltpu.make_async_copy(dst_hbm.at[pl.ds(wid * BPT, BPT)], didx_v,
                          isem.at[1]).wait()
    plsc.subcore_barrier()
    pltpu.async_copy(g_src(0), buf(0), gsem.at[0])
    pltpu.async_copy(g_src(1), buf(1), gsem.at[1])

    def body(t, _):
        j0 = 2 * t
        pltpu.make_async_copy(g_src(j0), buf(0), gsem.at[0]).wait()
        pltpu.async_copy(buf(0), s_dst(j0), ssem.at[0], add=True)

        @pl.when(t > 0)
        def _():
            pltpu.make_async_copy(buf(1), s_dst(j0 - 1), ssem.at[1]).wait()
            pltpu.async_copy(g_src(j0 + 1), buf(1), gsem.at[1])

        j1 = 2 * t + 1
        pltpu.make_async_copy(g_src(j1), buf(1), gsem.at[1]).wait()
        pltpu.async_copy(buf(1), s_dst(j1), ssem.at[1], add=True)
        pltpu.make_async_copy(buf(0), s_dst(j0), ssem.at[0]).wait()

        @pl.when(t < _NJ // 2 - 1)
        def _():
            pltpu.async_copy(g_src(j1 + 1), buf(0), gsem.at[0])
        return 0

    lax.fori_loop(0, _NJ // 2, body, 0)
    pltpu.make_async_copy(buf(1), s_dst(_NJ - 1), ssem.at[1]).wait()
    plsc.subcore_barrier()
    pltpu.sync_copy(acc_sh.at[pl.ds(base, RPT)],
                    out_hbm.at[cid, pl.ds(base, RPT)])


_SC_PARAMS = pltpu.CompilerParams(use_tc_tiling_on_sc=False)

_deg_call = pl.kernel(
    _sc_degree_body,
    out_type=jax.ShapeDtypeStruct((NC, NPAD, LANES), jnp.float32),
    mesh=_MESH,
    compiler_params=_SC_PARAMS,
    scratch_types=[
        pltpu.VMEM((BPT, EPB), jnp.int32),
        pltpu.VMEM((EPB, LANES), jnp.float32),
        pltpu.VMEM((EPB, LANES), jnp.float32),
        pltpu.VMEM_SHARED((NPAD, LANES), jnp.float32),
        pltpu.SemaphoreType.DMA,
    ],
)

_spmm_call = pl.kernel(
    _sc_spmm_body,
    out_type=jax.ShapeDtypeStruct((NC, NPAD, HID), jnp.float32),
    mesh=_MESH,
    compiler_params=_SC_PARAMS,
    scratch_types=[
        pltpu.VMEM((BPT, EPB), jnp.int32),
        pltpu.VMEM((BPT, EPB), jnp.int32),
        pltpu.VMEM((2 * EPB, HID), jnp.float32),
        pltpu.VMEM_SHARED((NPAD, HID), jnp.float32),
        pltpu.VMEM_SHARED((NPAD, HID), jnp.float32),
        pltpu.SemaphoreType.DMA((2,)),
        pltpu.SemaphoreType.DMA((2,)),
        pltpu.SemaphoreType.DMA((2,)),
    ],
)

_BM = 512
_GRID = NPAD // _BM


def _head_body(x_ref, w1_ref, degp_ref, dinv_ref, u1_ref):
    deg = degp_ref[0, :, 0:1] + degp_ref[1, :, 0:1] + 1.0
    dinv = lax.rsqrt(deg)
    dinv_ref[...] = dinv
    t = jnp.dot(x_ref[...], w1_ref[...], preferred_element_type=jnp.float32)
    u1_ref[...] = dinv * t


def _combine_body(n_mats, relu, pre_bias, post_scale, post_bias, *refs):
    i = 0
    acc_ref = refs[i]; i += 1
    u_ref = refs[i]; i += 1
    dinv_ref = refs[i]; i += 1
    b_ref = None
    if pre_bias:
        b_ref = refs[i]; i += 1
    w_refs = refs[i:i + n_mats]; i += n_mats
    pb_ref = None
    if post_bias:
        pb_ref = refs[i]; i += 1
    out_refs = refs[i:]

    dinv = dinv_ref[...]
    z = dinv * (acc_ref[0] + acc_ref[1] + u_ref[...])
    if pre_bias:
        z = z + b_ref[...]
    if relu:
        z = jnp.maximum(z, 0.0)
    if n_mats == 0:
        out_refs[0][...] = dinv * z if post_scale else z
    else:
        for w_ref, o_ref in zip(w_refs, out_refs):
            t = jnp.dot(z, w_ref[...], preferred_element_type=jnp.float32)
            if post_scale:
                t = dinv * t
            if post_bias:
                t = t + pb_ref[...]
            o_ref[...] = t


def _gram_body(a_ref, b_ref, o_ref):
    res = lax.dot_general(
        a_ref[...], b_ref[...], (((1,), (1,)), ((), ())),
        preferred_element_type=jnp.float32)
    o_ref[...] = res[:, :N_NODES]


def _row_spec(width):
    return pl.BlockSpec((_BM, width), lambda i: (i, 0))


def _full_spec(r, c):
    return pl.BlockSpec((r, c), lambda i: (0, 0))


_ACC_SPEC = pl.BlockSpec((NC, _BM, HID), lambda i: (0, i, 0))

_head_call = pl.pallas_call(
    _head_body,
    grid=(_GRID,),
    in_specs=[_row_spec(IN_DIM), _full_spec(IN_DIM, HID),
              pl.BlockSpec((NC, _BM, LANES), lambda i: (0, i, 0))],
    out_specs=[_row_spec(1), _row_spec(HID)],
    out_shape=[jax.ShapeDtypeStruct((NPAD, 1), jnp.float32),
               jax.ShapeDtypeStruct((NPAD, HID), jnp.float32)],
)


def _make_combine(n_mats, relu, pre_bias, post_scale, post_bias,
                  out_widths, out_rows):
    in_specs = [_ACC_SPEC, _row_spec(HID), _row_spec(1)]
    if pre_bias:
        in_specs.append(_full_spec(1, HID))
    for _ in range(n_mats):
        in_specs.append(_full_spec(HID, out_widths[0]))
    if post_bias:
        in_specs.append(_full_spec(1, out_widths[0]))
    return pl.pallas_call(
        functools.partial(_combine_body, n_mats, relu, pre_bias,
                          post_scale, post_bias),
        grid=(_GRID,),
        in_specs=in_specs,
        out_specs=[_row_spec(w) for w in out_widths],
        out_shape=[jax.ShapeDtypeStruct((out_rows, w), jnp.float32)
                   for w in out_widths],
    )


_s1_call = _make_combine(1, True, True, True, False, [HID], NPAD)
_s2p_call = _make_combine(0, False, True, True, False, [HID], NPAD)
_s4_call = _make_combine(1, False, False, False, True, [IN_DIM], N_NODES)

def _dec_body(acc_ref, u_ref, dinv_ref, w5_ref, b5_ref, w3_ref, b3_ref,
              h3_ref, u4_ref):
    dinv = dinv_ref[...]
    g = dinv * (acc_ref[0] + acc_ref[1] + u_ref[...])
    h3_ref[...] = jnp.dot(g, w5_ref[...],
                          preferred_element_type=jnp.float32) + b5_ref[...]
    h2 = jnp.maximum(jnp.dot(g, w3_ref[...],
                             preferred_element_type=jnp.float32) + b3_ref[...],
                     0.0)
    u4_ref[...] = dinv * h2


_dec_call = pl.pallas_call(
    _dec_body,
    grid=(_GRID,),
    in_specs=[_ACC_SPEC, _row_spec(HID), _row_spec(1),
              _full_spec(HID, HID), _full_spec(1, HID),
              _full_spec(HID, HID), _full_spec(1, HID)],
    out_specs=[_row_spec(HID), _row_spec(HID)],
    out_shape=[jax.ShapeDtypeStruct((NPAD, HID), jnp.float32),
               jax.ShapeDtypeStruct((NPAD, HID), jnp.float32)],
)

_GB = 256
_gram_call = pl.pallas_call(
    _gram_body,
    grid=(pl.cdiv(N_NODES, _GB),),
    in_specs=[pl.BlockSpec((_GB, HID), lambda i: (i, 0)),
              pl.BlockSpec((NPAD, HID), lambda i: (0, 0))],
    out_specs=pl.BlockSpec((_GB, N_NODES), lambda i: (i, 0)),
    out_shape=jax.ShapeDtypeStruct((N_NODES, N_NODES), jnp.float32),
)


def kernel(x, edge_index, W1, b1, W2, b2, W3, b3, W4, b4, W5, b5):
    ei = edge_index.astype(jnp.int32)
    pad = jnp.full((ETOT - N_EDGES,), N_NODES, jnp.int32)
    srcp = jnp.concatenate([ei[0], pad]).reshape(NW * BPT, EPB)
    dstp = jnp.concatenate([ei[1], pad]).reshape(NW * BPT, EPB)

    b1r = b1.reshape(1, HID)
    b2r = b2.reshape(1, HID)
    b3r = b3.reshape(1, HID)
    b4r = b4.reshape(1, IN_DIM)
    b5r = b5.reshape(1, HID)

    degp = _deg_call(dstp)
    dinv, u1 = _head_call(x, W1, degp)

    acc1 = _spmm_call(u1, srcp, dstp)
    (u2,) = _s1_call(acc1, u1, dinv, b1r, W2)

    acc2 = _spmm_call(u2, srcp, dstp)
    (ue,) = _s2p_call(acc2, u2, dinv, b2r)

    acce = _spmm_call(ue, srcp, dstp)
    h3, u4 = _dec_call(acce, ue, dinv, W5, b5r, W3, b3r)

    acc4 = _spmm_call(u4, srcp, dstp)
    (x_,) = _s4_call(acc4, u4, dinv, W4, b4r)

    s_ = _gram_call(h3, h3)
    return (x_, s_)


# speedup vs baseline: 27.7172x; 1.0037x over previous
import functools

import jax
import jax.numpy as jnp
from jax import lax
from jax.experimental import pallas as pl
from jax.experimental.pallas import tpu as pltpu
from jax.experimental.pallas import tpu_sc as plsc

N_NODES = 10000
N_EDGES = 320000
IN_DIM = 128
HID = 64

NC, NS, LANES = 2, 16, 16
NW = NC * NS
EPB = 128
BPT = 80
EPT = EPB * BPT
ETOT = EPT * NW
NPAD = 10240
RPT = NPAD // NS

_MESH = plsc.VectorSubcoreMesh(core_axis_name="c", subcore_axis_name="s")


def _sc_degree_body(dst_hbm, out_hbm, idx_v, ones_v, zrow_v, acc_sh, sem):
    cid = lax.axis_index("c")
    sid = lax.axis_index("s")
    wid = cid * NS + sid
    base = sid * RPT

    def fill(i, _):
        zrow_v[i] = jnp.zeros((LANES,), jnp.float32)
        ones_v[i] = jnp.ones((LANES,), jnp.float32)
        return 0

    lax.fori_loop(0, EPB, fill, 0)
    for k in range(RPT // EPB):
        pltpu.sync_copy(zrow_v, acc_sh.at[pl.ds(base + k * EPB, EPB)])
    pltpu.sync_copy(dst_hbm.at[pl.ds(wid * BPT, BPT)], idx_v)
    plsc.subcore_barrier()

    def fire(j, _):
        pltpu.async_copy(ones_v, acc_sh.at[idx_v.at[j]], sem, add=True)
        return 0

    def drain(j, _):
        pltpu.make_async_copy(ones_v, acc_sh.at[idx_v.at[j]], sem).wait()
        return 0

    lax.fori_loop(0, BPT, fire, 0)
    lax.fori_loop(0, BPT, drain, 0)
    plsc.subcore_barrier()
    pltpu.sync_copy(acc_sh.at[pl.ds(base, RPT)],
                    out_hbm.at[cid, pl.ds(base, RPT)])


_NJ = BPT


def _sc_spmm_body(tbl_hbm, src_hbm, dst_hbm, out_hbm,
                  sidx_v, didx_v, rows_v, acc_sh, tbl_sh,
                  gsem, ssem, isem):
    cid = lax.axis_index("c")
    sid = lax.axis_index("s")
    wid = cid * NS + sid
    base = sid * RPT

    def g_src(j):
        return tbl_sh.at[sidx_v.at[j]]

    def buf(slot):
        return rows_v.at[pl.ds(slot * EPB, EPB)]

    def s_dst(j):
        return acc_sh.at[didx_v.at[j]]

    pltpu.async_copy(src_hbm.at[pl.ds(wid * BPT, BPT)], sidx_v, isem.at[0])
    pltpu.async_copy(dst_hbm.at[pl.ds(wid * BPT, BPT)], didx_v, isem.at[1])

    def fill(i, _):
        for k in range(HID // LANES):
            rows_v[i, pl.ds(k * LANES, LANES)] = jnp.zeros((LANES,), jnp.float32)
        return 0

    lax.fori_loop(0, EPB, fill, 0)
    pltpu.sync_copy(tbl_hbm.at[pl.ds(base, RPT)], tbl_sh.at[pl.ds(base, RPT)])
    for k in range(RPT // EPB):
        pltpu.sync_copy(buf(0), acc_sh.at[pl.ds(base + k * EPB, EPB)])

    pltpu.make_async_copy(src_hbm.at[pl.ds(wid * BPT, BPT)], sidx_v,
                          isem.at[0]).wait()
    pltpu.make_async_copy(dst_hbm.at[pl.ds(wid * BPT, BPT)], didx_v,
                          isem.at[1]).wait()
    plsc.subcore_barrier()
    pltpu.async_copy(g_src(0), buf(0), gsem.at[0])
    pltpu.async_copy(g_src(1), buf(1), gsem.at[1])

    def body(t, _):
        j0 = 2 * t
        pltpu.make_async_copy(g_src(j0), buf(0), gsem.at[0]).wait()
        pltpu.async_copy(buf(0), s_dst(j0), ssem.at[0], add=True)

        @pl.when(t > 0)
        def _():
            pltpu.make_async_copy(buf(1), s_dst(j0 - 1), ssem.at[1]).wait()
            pltpu.async_copy(g_src(j0 + 1), buf(1), gsem.at[1])

        j1 = 2 * t + 1
        pltpu.make_async_copy(g_src(j1), buf(1), gsem.at[1]).wait()
        pltpu.async_copy(buf(1), s_dst(j1), ssem.at[1], add=True)
        pltpu.make_async_copy(buf(0), s_dst(j0), ssem.at[0]).wait()

        @pl.when(t < _NJ // 2 - 1)
        def _():
            pltpu.async_copy(g_src(j1 + 1), buf(0), gsem.at[0])
        return 0

    lax.fori_loop(0, _NJ // 2, body, 0)
    pltpu.make_async_copy(buf(1), s_dst(_NJ - 1), ssem.at[1]).wait()
    plsc.subcore_barrier()
    pltpu.sync_copy(acc_sh.at[pl.ds(base, RPT)],
                    out_hbm.at[cid, pl.ds(base, RPT)])


_SC_PARAMS = pltpu.CompilerParams(use_tc_tiling_on_sc=False)

_deg_call = pl.kernel(
    _sc_degree_body,
    out_type=jax.ShapeDtypeStruct((NC, NPAD, LANES), jnp.float32),
    mesh=_MESH,
    compiler_params=_SC_PARAMS,
    scratch_types=[
        pltpu.VMEM((BPT, EPB), jnp.int32),
        pltpu.VMEM((EPB, LANES), jnp.float32),
        pltpu.VMEM((EPB, LANES), jnp.float32),
        pltpu.VMEM_SHARED((NPAD, LANES), jnp.float32),
        pltpu.SemaphoreType.DMA,
    ],
)

_spmm_call = pl.kernel(
    _sc_spmm_body,
    out_type=jax.ShapeDtypeStruct((NC, NPAD, HID), jnp.float32),
    mesh=_MESH,
    compiler_params=_SC_PARAMS,
    scratch_types=[
        pltpu.VMEM((BPT, EPB), jnp.int32),
        pltpu.VMEM((BPT, EPB), jnp.int32),
        pltpu.VMEM((2 * EPB, HID), jnp.float32),
        pltpu.VMEM_SHARED((NPAD, HID), jnp.float32),
        pltpu.VMEM_SHARED((NPAD, HID), jnp.float32),
        pltpu.SemaphoreType.DMA((2,)),
        pltpu.SemaphoreType.DMA((2,)),
        pltpu.SemaphoreType.DMA((2,)),
    ],
)

_BM = 512
_GRID = NPAD // _BM


def _head_body(x_ref, w1_ref, degp_ref, dinv_ref, u1_ref):
    deg = degp_ref[0, :, 0:1] + degp_ref[1, :, 0:1] + 1.0
    dinv = lax.rsqrt(deg)
    dinv_ref[...] = dinv
    t = jnp.dot(x_ref[...], w1_ref[...], preferred_element_type=jnp.float32)
    u1_ref[...] = dinv * t


def _combine_body(n_mats, relu, pre_bias, post_scale, post_bias, *refs):
    i = 0
    acc_ref = refs[i]; i += 1
    u_ref = refs[i]; i += 1
    dinv_ref = refs[i]; i += 1
    b_ref = None
    if pre_bias:
        b_ref = refs[i]; i += 1
    w_refs = refs[i:i + n_mats]; i += n_mats
    pb_ref = None
    if post_bias:
        pb_ref = refs[i]; i += 1
    out_refs = refs[i:]

    dinv = dinv_ref[...]
    z = dinv * (acc_ref[0] + acc_ref[1] + u_ref[...])
    if pre_bias:
        z = z + b_ref[...]
    if relu:
        z = jnp.maximum(z, 0.0)
    if n_mats == 0:
        out_refs[0][...] = dinv * z if post_scale else z
    else:
        for w_ref, o_ref in zip(w_refs, out_refs):
            t = jnp.dot(z, w_ref[...], preferred_element_type=jnp.float32)
            if post_scale:
                t = dinv * t
            if post_bias:
                t = t + pb_ref[...]
            o_ref[...] = t


def _gram_body(a_ref, b_ref, o_ref):
    res = lax.dot_general(
        a_ref[...], b_ref[...], (((1,), (1,)), ((), ())),
        preferred_element_type=jnp.float32)
    o_ref[...] = res[:, :N_NODES]


def _row_spec(width):
    return pl.BlockSpec((_BM, width), lambda i: (i, 0))


def _full_spec(r, c):
    return pl.BlockSpec((r, c), lambda i: (0, 0))


_ACC_SPEC = pl.BlockSpec((NC, _BM, HID), lambda i: (0, i, 0))

_head_call = pl.pallas_call(
    _head_body,
    grid=(_GRID,),
    in_specs=[_row_spec(IN_DIM), _full_spec(IN_DIM, HID),
              pl.BlockSpec((NC, _BM, LANES), lambda i: (0, i, 0))],
    out_specs=[_row_spec(1), _row_spec(HID)],
    out_shape=[jax.ShapeDtypeStruct((NPAD, 1), jnp.float32),
               jax.ShapeDtypeStruct((NPAD, HID), jnp.float32)],
)


def _make_combine(n_mats, relu, pre_bias, post_scale, post_bias,
                  out_widths, out_rows):
    in_specs = [_ACC_SPEC, _row_spec(HID), _row_spec(1)]
    if pre_bias:
        in_specs.append(_full_spec(1, HID))
    for _ in range(n_mats):
        in_specs.append(_full_spec(HID, out_widths[0]))
    if post_bias:
        in_specs.append(_full_spec(1, out_widths[0]))
    return pl.pallas_call(
        functools.partial(_combine_body, n_mats, relu, pre_bias,
                          post_scale, post_bias),
        grid=(_GRID,),
        in_specs=in_specs,
        out_specs=[_row_spec(w) for w in out_widths],
        out_shape=[jax.ShapeDtypeStruct((out_rows, w), jnp.float32)
                   for w in out_widths],
    )


_s1_call = _make_combine(1, True, True, True, False, [HID], NPAD)
_s2p_call = _make_combine(0, False, True, True, False, [HID], NPAD)
_s4_call = _make_combine(1, False, False, False, True, [IN_DIM], N_NODES)

def _dec_body(acc_ref, u_ref, dinv_ref, w5_ref, b5_ref, w3_ref, b3_ref,
              h3_ref, u4_ref):
    dinv = dinv_ref[...]
    g = dinv * (acc_ref[0] + acc_ref[1] + u_ref[...])
    h3f = jnp.dot(g, w5_ref[...],
                  preferred_element_type=jnp.float32) + b5_ref[...]
    h3_ref[...] = h3f.astype(jnp.bfloat16)
    h2 = jnp.maximum(jnp.dot(g, w3_ref[...],
                             preferred_element_type=jnp.float32) + b3_ref[...],
                     0.0)
    u4_ref[...] = dinv * h2


_dec_call = pl.pallas_call(
    _dec_body,
    grid=(_GRID,),
    in_specs=[_ACC_SPEC, _row_spec(HID), _row_spec(1),
              _full_spec(HID, HID), _full_spec(1, HID),
              _full_spec(HID, HID), _full_spec(1, HID)],
    out_specs=[_row_spec(HID), _row_spec(HID)],
    out_shape=[jax.ShapeDtypeStruct((NPAD, HID), jnp.bfloat16),
               jax.ShapeDtypeStruct((NPAD, HID), jnp.float32)],
)

_GB = 256
_gram_call = pl.pallas_call(
    _gram_body,
    grid=(pl.cdiv(N_NODES, _GB),),
    in_specs=[pl.BlockSpec((_GB, HID), lambda i: (i, 0)),
              pl.BlockSpec((NPAD, HID), lambda i: (0, 0))],
    out_specs=pl.BlockSpec((_GB, N_NODES), lambda i: (i, 0)),
    out_shape=jax.ShapeDtypeStruct((N_NODES, N_NODES), jnp.float32),
)


def kernel(x, edge_index, W1, b1, W2, b2, W3, b3, W4, b4, W5, b5):
    ei = edge_index.astype(jnp.int32)
    pad = jnp.full((ETOT - N_EDGES,), N_NODES, jnp.int32)
    srcp = jnp.concatenate([ei[0], pad]).reshape(NW * BPT, EPB)
    dstp = jnp.concatenate([ei[1], pad]).reshape(NW * BPT, EPB)

    b1r = b1.reshape(1, HID)
    b2r = b2.reshape(1, HID)
    b3r = b3.reshape(1, HID)
    b4r = b4.reshape(1, IN_DIM)
    b5r = b5.reshape(1, HID)

    degp = _deg_call(dstp)
    dinv, u1 = _head_call(x, W1, degp)

    acc1 = _spmm_call(u1, srcp, dstp)
    (u2,) = _s1_call(acc1, u1, dinv, b1r, W2)

    acc2 = _spmm_call(u2, srcp, dstp)
    (ue,) = _s2p_call(acc2, u2, dinv, b2r)

    acce = _spmm_call(ue, srcp, dstp)
    h3, u4 = _dec_call(acce, ue, dinv, W5, b5r, W3, b3r)

    acc4 = _spmm_call(u4, srcp, dstp)
    (x_,) = _s4_call(acc4, u4, dinv, W4, b4r)

    s_ = _gram_call(h3, h3)
    return (x_, s_)


# speedup vs baseline: 27.7221x; 1.0002x over previous
import functools

import jax
import jax.numpy as jnp
from jax import lax
from jax.experimental import pallas as pl
from jax.experimental.pallas import tpu as pltpu
from jax.experimental.pallas import tpu_sc as plsc

N_NODES = 10000
N_EDGES = 320000
IN_DIM = 128
HID = 64

NC, NS, LANES = 2, 16, 16
NW = NC * NS
EPB = 128
BPT = 80
EPT = EPB * BPT
ETOT = EPT * NW
NPAD = 10240
RPT = NPAD // NS

_MESH = plsc.VectorSubcoreMesh(core_axis_name="c", subcore_axis_name="s")


def _sc_degree_body(dst_hbm, out_hbm, idx_v, ones_v, zrow_v, acc_sh, sem):
    cid = lax.axis_index("c")
    sid = lax.axis_index("s")
    wid = cid * NS + sid
    base = sid * RPT

    def fill(i, _):
        zrow_v[i] = jnp.zeros((LANES,), jnp.float32)
        ones_v[i] = jnp.ones((LANES,), jnp.float32)
        return 0

    lax.fori_loop(0, EPB, fill, 0)
    for k in range(RPT // EPB):
        pltpu.sync_copy(zrow_v, acc_sh.at[pl.ds(base + k * EPB, EPB)])
    pltpu.sync_copy(dst_hbm.at[pl.ds(wid * BPT, BPT)], idx_v)
    plsc.subcore_barrier()

    def fire(j, _):
        pltpu.async_copy(ones_v, acc_sh.at[idx_v.at[j]], sem, add=True)
        return 0

    def drain(j, _):
        pltpu.make_async_copy(ones_v, acc_sh.at[idx_v.at[j]], sem).wait()
        return 0

    lax.fori_loop(0, BPT, fire, 0)
    lax.fori_loop(0, BPT, drain, 0)
    plsc.subcore_barrier()
    pltpu.sync_copy(acc_sh.at[pl.ds(base, RPT)],
                    out_hbm.at[cid, pl.ds(base, RPT)])


_NJ = BPT


def _sc_spmm_body(tbl_hbm, src_hbm, dst_hbm, out_hbm,
                  sidx_v, didx_v, rows_v, acc_sh, tbl_sh,
                  gsem, ssem, isem):
    cid = lax.axis_index("c")
    sid = lax.axis_index("s")
    wid = cid * NS + sid
    base = sid * RPT

    def g_src(j):
        return tbl_sh.at[sidx_v.at[j]]

    def buf(slot):
        return rows_v.at[pl.ds(slot * EPB, EPB)]

    def s_dst(j):
        return acc_sh.at[didx_v.at[j]]

    pltpu.async_copy(src_hbm.at[pl.ds(wid * BPT, BPT)], sidx_v, isem.at[0])
    pltpu.async_copy(dst_hbm.at[pl.ds(wid * BPT, BPT)], didx_v, isem.at[1])

    def fill(i, _):
        for k in range(HID // LANES):
            rows_v[i, pl.ds(k * LANES, LANES)] = jnp.zeros((LANES,), jnp.float32)
        return 0

    lax.fori_loop(0, EPB, fill, 0)
    pltpu.sync_copy(tbl_hbm.at[pl.ds(base, RPT)], tbl_sh.at[pl.ds(base, RPT)])
    for k in range(RPT // EPB):
        pltpu.sync_copy(buf(0), acc_sh.at[pl.ds(base + k * EPB, EPB)])

    pltpu.make_async_copy(src_hbm.at[pl.ds(wid * BPT, BPT)], sidx_v,
                          isem.at[0]).wait()
    pltpu.make_async_copy(dst_hbm.at[pl.ds(wid * BPT, BPT)], didx_v,
                          isem.at[1]).wait()
    plsc.subcore_barrier()
    pltpu.async_copy(g_src(0), buf(0), gsem.at[0])
    pltpu.async_copy(g_src(1), buf(1), gsem.at[1])

    def body(t, _):
        j0 = 2 * t
        pltpu.make_async_copy(g_src(j0), buf(0), gsem.at[0]).wait()
        pltpu.async_copy(buf(0), s_dst(j0), ssem.at[0], add=True)

        @pl.when(t > 0)
        def _():
            pltpu.make_async_copy(buf(1), s_dst(j0 - 1), ssem.at[1]).wait()
            pltpu.async_copy(g_src(j0 + 1), buf(1), gsem.at[1])

        j1 = 2 * t + 1
        pltpu.make_async_copy(g_src(j1), buf(1), gsem.at[1]).wait()
        pltpu.async_copy(buf(1), s_dst(j1), ssem.at[1], add=True)
        pltpu.make_async_copy(buf(0), s_dst(j0), ssem.at[0]).wait()

        @pl.when(t < _NJ // 2 - 1)
        def _():
            pltpu.async_copy(g_src(j1 + 1), buf(0), gsem.at[0])
        return 0

    lax.fori_loop(0, _NJ // 2, body, 0)
    pltpu.make_async_copy(buf(1), s_dst(_NJ - 1), ssem.at[1]).wait()
    plsc.subcore_barrier()
    pltpu.sync_copy(acc_sh.at[pl.ds(base, RPT)],
                    out_hbm.at[cid, pl.ds(base, RPT)])


_SC_PARAMS = pltpu.CompilerParams(use_tc_tiling_on_sc=False)

_deg_call = pl.kernel(
    _sc_degree_body,
    out_type=jax.ShapeDtypeStruct((NC, NPAD, LANES), jnp.float32),
    mesh=_MESH,
    compiler_params=_SC_PARAMS,
    scratch_types=[
        pltpu.VMEM((BPT, EPB), jnp.int32),
        pltpu.VMEM((EPB, LANES), jnp.float32),
        pltpu.VMEM((EPB, LANES), jnp.float32),
        pltpu.VMEM_SHARED((NPAD, LANES), jnp.float32),
        pltpu.SemaphoreType.DMA,
    ],
)

_spmm_call = pl.kernel(
    _sc_spmm_body,
    out_type=jax.ShapeDtypeStruct((NC, NPAD, HID), jnp.float32),
    mesh=_MESH,
    compiler_params=_SC_PARAMS,
    scratch_types=[
        pltpu.VMEM((BPT, EPB), jnp.int32),
        pltpu.VMEM((BPT, EPB), jnp.int32),
        pltpu.VMEM((2 * EPB, HID), jnp.float32),
        pltpu.VMEM_SHARED((NPAD, HID), jnp.float32),
        pltpu.VMEM_SHARED((NPAD, HID), jnp.float32),
        pltpu.SemaphoreType.DMA((2,)),
        pltpu.SemaphoreType.DMA((2,)),
        pltpu.SemaphoreType.DMA((2,)),
    ],
)

_BM = 512
_GRID = NPAD // _BM


def _mm1_body(x_ref, w1_ref, t1_ref):
    t1_ref[...] = jnp.dot(x_ref[...], w1_ref[...],
                          preferred_element_type=jnp.float32)


def _head_body(t1_ref, degp_ref, dinv_ref, u1_ref):
    deg = degp_ref[0, :, 0:1] + degp_ref[1, :, 0:1] + 1.0
    dinv = lax.rsqrt(deg)
    dinv_ref[...] = dinv
    u1_ref[...] = dinv * t1_ref[...]


def _combine_body(n_mats, relu, pre_bias, post_scale, post_bias, *refs):
    i = 0
    acc_ref = refs[i]; i += 1
    u_ref = refs[i]; i += 1
    dinv_ref = refs[i]; i += 1
    b_ref = None
    if pre_bias:
        b_ref = refs[i]; i += 1
    w_refs = refs[i:i + n_mats]; i += n_mats
    pb_ref = None
    if post_bias:
        pb_ref = refs[i]; i += 1
    out_refs = refs[i:]

    dinv = dinv_ref[...]
    z = dinv * (acc_ref[0] + acc_ref[1] + u_ref[...])
    if pre_bias:
        z = z + b_ref[...]
    if relu:
        z = jnp.maximum(z, 0.0)
    if n_mats == 0:
        out_refs[0][...] = dinv * z if post_scale else z
    else:
        for w_ref, o_ref in zip(w_refs, out_refs):
            t = jnp.dot(z, w_ref[...], preferred_element_type=jnp.float32)
            if post_scale:
                t = dinv * t
            if post_bias:
                t = t + pb_ref[...]
            o_ref[...] = t


def _gram_body(a_ref, b_ref, o_ref):
    res = lax.dot_general(
        a_ref[...], b_ref[...], (((1,), (1,)), ((), ())),
        preferred_element_type=jnp.float32)
    o_ref[...] = res[:, :N_NODES]


def _row_spec(width):
    return pl.BlockSpec((_BM, width), lambda i: (i, 0))


def _full_spec(r, c):
    return pl.BlockSpec((r, c), lambda i: (0, 0))


_ACC_SPEC = pl.BlockSpec((NC, _BM, HID), lambda i: (0, i, 0))

_mm1_call = pl.pallas_call(
    _mm1_body,
    grid=(_GRID,),
    in_specs=[_row_spec(IN_DIM), _full_spec(IN_DIM, HID)],
    out_specs=_row_spec(HID),
    out_shape=jax.ShapeDtypeStruct((NPAD, HID), jnp.float32),
)

_head_call = pl.pallas_call(
    _head_body,
    grid=(_GRID,),
    in_specs=[_row_spec(HID),
              pl.BlockSpec((NC, _BM, LANES), lambda i: (0, i, 0))],
    out_specs=[_row_spec(1), _row_spec(HID)],
    out_shape=[jax.ShapeDtypeStruct((NPAD, 1), jnp.float32),
               jax.ShapeDtypeStruct((NPAD, HID), jnp.float32)],
)


def _make_combine(n_mats, relu, pre_bias, post_scale, post_bias,
                  out_widths, out_rows):
    in_specs = [_ACC_SPEC, _row_spec(HID), _row_spec(1)]
    if pre_bias:
        in_specs.append(_full_spec(1, HID))
    for _ in range(n_mats):
        in_specs.append(_full_spec(HID, out_widths[0]))
    if post_bias:
        in_specs.append(_full_spec(1, out_widths[0]))
    return pl.pallas_call(
        functools.partial(_combine_body, n_mats, relu, pre_bias,
                          post_scale, post_bias),
        grid=(_GRID,),
        in_specs=in_specs,
        out_specs=[_row_spec(w) for w in out_widths],
        out_shape=[jax.ShapeDtypeStruct((out_rows, w), jnp.float32)
                   for w in out_widths],
    )


_s1_call = _make_combine(1, True, True, True, False, [HID], NPAD)
_s2p_call = _make_combine(0, False, True, True, False, [HID], NPAD)
_s4_call = _make_combine(1, False, False, False, True, [IN_DIM], N_NODES)

def _dec_body(acc_ref, u_ref, dinv_ref, w5_ref, b5_ref, w3_ref, b3_ref,
              h3_ref, u4_ref):
    dinv = dinv_ref[...]
    g = dinv * (acc_ref[0] + acc_ref[1] + u_ref[...])
    h3f = jnp.dot(g, w5_ref[...],
                  preferred_element_type=jnp.float32) + b5_ref[...]
    h3_ref[...] = h3f.astype(jnp.bfloat16)
    h2 = jnp.maximum(jnp.dot(g, w3_ref[...],
                             preferred_element_type=jnp.float32) + b3_ref[...],
                     0.0)
    u4_ref[...] = dinv * h2


_dec_call = pl.pallas_call(
    _dec_body,
    grid=(_GRID,),
    in_specs=[_ACC_SPEC, _row_spec(HID), _row_spec(1),
              _full_spec(HID, HID), _full_spec(1, HID),
              _full_spec(HID, HID), _full_spec(1, HID)],
    out_specs=[_row_spec(HID), _row_spec(HID)],
    out_shape=[jax.ShapeDtypeStruct((NPAD, HID), jnp.bfloat16),
               jax.ShapeDtypeStruct((NPAD, HID), jnp.float32)],
)

_GB = 256
_gram_call = pl.pallas_call(
    _gram_body,
    grid=(pl.cdiv(N_NODES, _GB),),
    in_specs=[pl.BlockSpec((_GB, HID), lambda i: (i, 0)),
              pl.BlockSpec((NPAD, HID), lambda i: (0, 0))],
    out_specs=pl.BlockSpec((_GB, N_NODES), lambda i: (i, 0)),
    out_shape=jax.ShapeDtypeStruct((N_NODES, N_NODES), jnp.float32),
)


def kernel(x, edge_index, W1, b1, W2, b2, W3, b3, W4, b4, W5, b5):
    ei = edge_index.astype(jnp.int32)
    pad = jnp.full((ETOT - N_EDGES,), N_NODES, jnp.int32)
    srcp = jnp.concatenate([ei[0], pad]).reshape(NW * BPT, EPB)
    dstp = jnp.concatenate([ei[1], pad]).reshape(NW * BPT, EPB)

    b1r = b1.reshape(1, HID)
    b2r = b2.reshape(1, HID)
    b3r = b3.reshape(1, HID)
    b4r = b4.reshape(1, IN_DIM)
    b5r = b5.reshape(1, HID)

    degp = _deg_call(dstp)
    t1 = _mm1_call(x, W1)
    dinv, u1 = _head_call(t1, degp)

    acc1 = _spmm_call(u1, srcp, dstp)
    (u2,) = _s1_call(acc1, u1, dinv, b1r, W2)

    acc2 = _spmm_call(u2, srcp, dstp)
    (ue,) = _s2p_call(acc2, u2, dinv, b2r)

    acce = _spmm_call(ue, srcp, dstp)
    h3, u4 = _dec_call(acce, ue, dinv, W5, b5r, W3, b3r)

    acc4 = _spmm_call(u4, srcp, dstp)
    (x_,) = _s4_call(acc4, u4, dinv, W4, b4r)

    s_ = _gram_call(h3, h3)
    return (x_, s_)
